# baseline (speedup 1.0000x reference)
import functools

import jax
import jax.numpy as jnp
from jax import lax
from jax.experimental import pallas as pl
from jax.experimental.pallas import tpu as pltpu
from jax.experimental.pallas import tpu_sc as plsc

VOCAB = 1000
D = 128
B_TOTAL = 1024 * 50

_info = plsc.get_sparse_core_info()
NC = _info.num_cores
NS = _info.num_subcores
NW = NC * NS
B_PER_W = B_TOTAL // NW
CHUNK = 50
NCHUNK = B_PER_W // CHUNK


def _table_body(w_ref, b_ref, out_ref):
    w = w_ref[...]
    r = lax.broadcasted_iota(jnp.int32, (D, D), 0)
    c = lax.broadcasted_iota(jnp.int32, (D, D), 1)
    eye = jnp.where(r == c, 1.0, 0.0).astype(jnp.float32)
    t = lax.dot_general(
        w, eye,
        dimension_numbers=(((0,), (0,)), ((), ())),
        preferred_element_type=jnp.float32,
    )
    out_ref[...] = t + b_ref[...]


def _build_table(W, b):
    return pl.pallas_call(
        _table_body,
        out_shape=jax.ShapeDtypeStruct((VOCAB, D), jnp.float32),
    )(W, b.reshape(1, D))


NBUF = 8


def _sc_body(table_hbm, idx_hbm, out_hbm, idx_v, buf_v, *sems):
    sg = sems[:NBUF]
    ss = sems[NBUF:]
    wid = lax.axis_index("s") * NC + lax.axis_index("c")
    pltpu.sync_copy(idx_hbm.at[wid], idx_v)

    gh = [None] * NCHUNK
    sh = [None] * NCHUNK
    s_waited = [False] * NCHUNK

    def gather(g):
        gh[g] = pltpu.async_copy(
            table_hbm.at[idx_v.at[g]], buf_v.at[g % NBUF], sg[g % NBUF])

    def scatter(j):
        sh[j] = pltpu.async_copy(
            buf_v.at[j % NBUF], out_hbm.at[wid * NCHUNK + j], ss[j % NBUF])

    for g in range(min(NBUF - 1, NCHUNK)):
        gather(g)
    for j in range(NCHUNK):
        gh[j].wait()
        scatter(j)
        g = j + NBUF - 1
        if g < NCHUNK:
            if j >= 1:
                sh[j - 1].wait()
                s_waited[j - 1] = True
            gather(g)
    for j in range(NCHUNK):
        if not s_waited[j]:
            sh[j].wait()


def _sc_lookup(table, idx):
    mesh = plsc.VectorSubcoreMesh(core_axis_name="c", subcore_axis_name="s")
    k = pl.kernel(
        _sc_body,
        mesh=mesh,
        out_type=jax.ShapeDtypeStruct((NW * NCHUNK, CHUNK, D), jnp.float32),
        scratch_types=[
            pltpu.VMEM((NCHUNK, CHUNK), jnp.int32),
            pltpu.VMEM((NBUF, CHUNK, D), jnp.float32),
        ] + [pltpu.SemaphoreType.DMA] * (2 * NBUF),
        compiler_params=pltpu.CompilerParams(use_tc_tiling_on_sc=True),
    )
    return k(table, idx)


def kernel(x, W, b):
    idx = x.astype(jnp.int32).reshape(NW, NCHUNK, CHUNK)
    table = _build_table(W, b)
    out = _sc_lookup(table, idx)
    return out.reshape(x.shape[0], x.shape[1], D)


# speedup vs baseline: 1.3029x; 1.3029x over previous
import functools

import jax
import jax.numpy as jnp
from jax import lax
from jax.experimental import pallas as pl
from jax.experimental.pallas import tpu as pltpu
from jax.experimental.pallas import tpu_sc as plsc

VOCAB = 1000
D = 128
B_TOTAL = 1024 * 50

_info = plsc.get_sparse_core_info()
NC = _info.num_cores
NS = _info.num_subcores
NW = NC * NS
B_PER_W = B_TOTAL // NW
CHUNK = 80
NCHUNK = B_PER_W // CHUNK


def _table_body(w_ref, b_ref, out_ref):
    w = w_ref[...]
    r = lax.broadcasted_iota(jnp.int32, (D, D), 0)
    c = lax.broadcasted_iota(jnp.int32, (D, D), 1)
    eye = jnp.where(r == c, 1.0, 0.0).astype(jnp.float32)
    t = lax.dot_general(
        w, eye,
        dimension_numbers=(((0,), (0,)), ((), ())),
        preferred_element_type=jnp.float32,
    )
    out_ref[...] = t + b_ref[...]


def _build_table(W, b):
    return pl.pallas_call(
        _table_body,
        out_shape=jax.ShapeDtypeStruct((VOCAB, D), jnp.float32),
    )(W, b.reshape(1, D))


NBUF = 8


def _sc_body(table_hbm, idx_hbm, out_hbm, idx_v, buf_v, *sems):
    sg = sems[:NBUF]
    ss = sems[NBUF:]
    wid = lax.axis_index("s") * NC + lax.axis_index("c")
    pltpu.sync_copy(idx_hbm.at[wid], idx_v)

    gh = [None] * NCHUNK
    sh = [None] * NCHUNK
    s_waited = [False] * NCHUNK

    def gather(g):
        gh[g] = pltpu.async_copy(
            table_hbm.at[idx_v.at[g]], buf_v.at[g % NBUF], sg[g % NBUF])

    def scatter(j):
        sh[j] = pltpu.async_copy(
            buf_v.at[j % NBUF],
            out_hbm.at[pl.ds(wid * B_PER_W + j * CHUNK, CHUNK)],
            ss[j % NBUF])

    for g in range(min(NBUF - 1, NCHUNK)):
        gather(g)
    for j in range(NCHUNK):
        gh[j].wait()
        scatter(j)
        g = j + NBUF - 1
        if g < NCHUNK:
            if j >= 1:
                sh[j - 1].wait()
                s_waited[j - 1] = True
            gather(g)
    for j in range(NCHUNK):
        if not s_waited[j]:
            sh[j].wait()


def _sc_lookup(table, idx):
    mesh = plsc.VectorSubcoreMesh(core_axis_name="c", subcore_axis_name="s")
    k = pl.kernel(
        _sc_body,
        mesh=mesh,
        out_type=jax.ShapeDtypeStruct((B_TOTAL, D), jnp.float32),
        scratch_types=[
            pltpu.VMEM((NCHUNK, CHUNK), jnp.int32),
            pltpu.VMEM((NBUF, CHUNK, D), jnp.float32),
        ] + [pltpu.SemaphoreType.DMA] * (2 * NBUF),
        compiler_params=pltpu.CompilerParams(use_tc_tiling_on_sc=True),
    )
    return k(table, idx)


def kernel(x, W, b):
    bsz, seq = x.shape
    idx = x.astype(jnp.int32).T.reshape(NW, NCHUNK, CHUNK)
    table = _build_table(W, b)
    out = _sc_lookup(table, idx)
    return out.reshape(seq, bsz, D).transpose(1, 0, 2)


# speedup vs baseline: 1.3181x; 1.0117x over previous
import functools

import jax
import jax.numpy as jnp
from jax import lax
from jax.experimental import pallas as pl
from jax.experimental.pallas import tpu as pltpu
from jax.experimental.pallas import tpu_sc as plsc

VOCAB = 1000
D = 128
B_TOTAL = 1024 * 50

_info = plsc.get_sparse_core_info()
NC = _info.num_cores
NS = _info.num_subcores
NW = NC * NS
B_PER_W = B_TOTAL // NW
CHUNK = 80
NCHUNK = B_PER_W // CHUNK


def _table_body(w_ref, b_ref, out_ref):
    w = w_ref[...]
    r = lax.broadcasted_iota(jnp.int32, (D, D), 0)
    c = lax.broadcasted_iota(jnp.int32, (D, D), 1)
    eye = jnp.where(r == c, 1.0, 0.0).astype(jnp.float32)
    t = lax.dot_general(
        w, eye,
        dimension_numbers=(((0,), (0,)), ((), ())),
        preferred_element_type=jnp.float32,
    )
    out_ref[...] = t + b_ref[...]


def _build_table(W, b):
    return pl.pallas_call(
        _table_body,
        out_shape=jax.ShapeDtypeStruct((VOCAB, D), jnp.float32),
    )(W, b.reshape(1, D))


NBUF = 8
GDEPTH = 4


def _sc_body(table_hbm, idx_hbm, out_hbm, idx_v, buf_v, *sems):
    sg = sems[:NBUF]
    ss = sems[NBUF:]
    wid = lax.axis_index("s") * NC + lax.axis_index("c")
    pltpu.sync_copy(idx_hbm.at[wid], idx_v)

    gh = [None] * NCHUNK
    sh = [None] * NCHUNK
    s_waited = [False] * NCHUNK

    def gather(g):
        gh[g] = pltpu.async_copy(
            table_hbm.at[idx_v.at[g]], buf_v.at[g % NBUF], sg[g % NBUF])

    def scatter(j):
        sh[j] = pltpu.async_copy(
            buf_v.at[j % NBUF],
            out_hbm.at[pl.ds(wid * B_PER_W + j * CHUNK, CHUNK)],
            ss[j % NBUF])

    for g in range(min(GDEPTH, NCHUNK)):
        gather(g)
    for j in range(NCHUNK):
        gh[j].wait()
        scatter(j)
        nxt = j + GDEPTH
        if nxt < NCHUNK:
            prev_user = nxt - NBUF
            if prev_user >= 0:
                sh[prev_user].wait()
                s_waited[prev_user] = True
            gather(nxt)
    for j in range(NCHUNK):
        if not s_waited[j]:
            sh[j].wait()


def _sc_lookup(table, idx):
    mesh = plsc.VectorSubcoreMesh(core_axis_name="c", subcore_axis_name="s")
    k = pl.kernel(
        _sc_body,
        mesh=mesh,
        out_type=jax.ShapeDtypeStruct((B_TOTAL, D), jnp.float32),
        scratch_types=[
            pltpu.VMEM((NCHUNK, CHUNK), jnp.int32),
            pltpu.VMEM((NBUF, CHUNK, D), jnp.float32),
        ] + [pltpu.SemaphoreType.DMA] * (2 * NBUF),
        compiler_params=pltpu.CompilerParams(use_tc_tiling_on_sc=True),
    )
    return k(table, idx)


def kernel(x, W, b):
    bsz, seq = x.shape
    idx = x.astype(jnp.int32).T.reshape(NW, NCHUNK, CHUNK)
    table = _build_table(W, b)
    out = _sc_lookup(table, idx)
    return out.reshape(seq, bsz, D).transpose(1, 0, 2)


# speedup vs baseline: 2.0633x; 1.5654x over previous
import jax
import jax.numpy as jnp
from jax import lax
from jax.experimental import pallas as pl
from jax.experimental.pallas import tpu as pltpu
from jax.experimental.pallas import tpu_sc as plsc

VOCAB = 1000
D = 128
B_TOTAL = 1024 * 50

_info = plsc.get_sparse_core_info()
NC = _info.num_cores
NS = _info.num_subcores
NW = NC * NS
B_PER_W = B_TOTAL // NW
CHUNK = 80
NCHUNK = B_PER_W // CHUNK

NBUF = 8
GDEPTH = 4


def _table_body(w_ref, b_ref, out_ref):
    w = w_ref[...]
    r = lax.broadcasted_iota(jnp.int32, (D, D), 0)
    c = lax.broadcasted_iota(jnp.int32, (D, D), 1)
    eye = jnp.where(r == c, 1.0, 0.0).astype(jnp.float32)
    t = lax.dot_general(
        w, eye,
        dimension_numbers=(((0,), (0,)), ((), ())),
        preferred_element_type=jnp.float32,
    )
    out_ref[...] = t + b_ref[...]


def _build_table(W, b):
    return pl.pallas_call(
        _table_body,
        out_shape=jax.ShapeDtypeStruct((VOCAB, D), jnp.float32),
    )(W, b.reshape(1, D))


def _sc_body(table_hbm, idx_hbm, out_hbm, table_s, idx_v, buf_v, *sems):
    sg = sems[:NBUF]
    ss = sems[NBUF:]
    wid = lax.axis_index("s") * NC + lax.axis_index("c")

    @pl.when(lax.axis_index("s") == 0)
    def _load_table():
        pltpu.sync_copy(table_hbm, table_s)

    pltpu.sync_copy(idx_hbm.at[wid], idx_v)
    plsc.subcore_barrier()

    gh = [None] * NCHUNK
    sh = [None] * NCHUNK
    s_waited = [False] * NCHUNK

    def gather(g):
        gh[g] = pltpu.async_copy(
            table_s.at[idx_v.at[g]], buf_v.at[g % NBUF], sg[g % NBUF])

    def scatter(j):
        sh[j] = pltpu.async_copy(
            buf_v.at[j % NBUF],
            out_hbm.at[pl.ds(wid * B_PER_W + j * CHUNK, CHUNK)],
            ss[j % NBUF])

    for g in range(min(GDEPTH, NCHUNK)):
        gather(g)
    for j in range(NCHUNK):
        gh[j].wait()
        scatter(j)
        nxt = j + GDEPTH
        if nxt < NCHUNK:
            prev_user = nxt - NBUF
            if prev_user >= 0:
                sh[prev_user].wait()
                s_waited[prev_user] = True
            gather(nxt)
    for j in range(NCHUNK):
        if not s_waited[j]:
            sh[j].wait()


def _sc_lookup(table, idx):
    mesh = plsc.VectorSubcoreMesh(core_axis_name="c", subcore_axis_name="s")
    k = pl.kernel(
        _sc_body,
        mesh=mesh,
        out_type=jax.ShapeDtypeStruct((B_TOTAL, D), jnp.float32),
        scratch_types=[
            pltpu.VMEM_SHARED((VOCAB, D), jnp.float32),
            pltpu.VMEM((NCHUNK, CHUNK), jnp.int32),
            pltpu.VMEM((NBUF, CHUNK, D), jnp.float32),
        ] + [pltpu.SemaphoreType.DMA] * (2 * NBUF),
        compiler_params=pltpu.CompilerParams(use_tc_tiling_on_sc=True),
    )
    return k(table, idx)


def kernel(x, W, b):
    bsz, seq = x.shape
    idx = x.astype(jnp.int32).T.reshape(NW, NCHUNK, CHUNK)
    table = _build_table(W, b)
    out = _sc_lookup(table, idx)
    return out.reshape(seq, bsz, D).transpose(1, 0, 2)
